# Initial kernel scaffold; baseline (speedup 1.0000x reference)
import jax
import jax.numpy as jnp
from jax.experimental import pallas as pl


def kernel(inputs, codebook):
    raise NotImplementedError("write your pallas kernel here")


# speedup vs baseline: 1.1821x; 1.1821x over previous
import functools

import jax
import jax.numpy as jnp
from jax import lax
from jax.experimental import pallas as pl
from jax.experimental.pallas import tpu as pltpu
from jax.experimental.pallas import tpu_sc as plsc

NK = 8192
D = 256
N = 8192
TM = 256
GRID = N // TM


def _dist_argmin_body(x_ref, cb_ref, csq_ref, idx_ref, dmin_ref):
    x = x_ref[...]
    xsq = jnp.sum(x * x, axis=1, keepdims=True)
    m = lax.dot_general(x, cb_ref[...], (((1,), (1,)), ((), ())),
                        preferred_element_type=jnp.float32)
    d = (xsq + csq_ref[...]) - 2.0 * m
    dmin = jnp.min(d, axis=1, keepdims=True)
    kio = lax.broadcasted_iota(jnp.int32, d.shape, 1)
    idx = jnp.min(jnp.where(d == dmin, kio, NK), axis=1)
    idx_ref[0, 0, :] = idx
    dmin_ref[0, 0, :] = dmin[:, 0]


def _dist_argmin(x, codebook, csq):
    return pl.pallas_call(
        _dist_argmin_body,
        grid=(GRID,),
        in_specs=[
            pl.BlockSpec((TM, D), lambda i: (i, 0)),
            pl.BlockSpec((NK, D), lambda i: (0, 0)),
            pl.BlockSpec((1, NK), lambda i: (0, 0)),
        ],
        out_specs=[
            pl.BlockSpec((1, 1, TM), lambda i: (i, 0, 0)),
            pl.BlockSpec((1, 1, TM), lambda i: (i, 0, 0)),
        ],
        out_shape=[
            jax.ShapeDtypeStruct((GRID, 1, TM), jnp.int32),
            jax.ShapeDtypeStruct((GRID, 1, TM), jnp.float32),
        ],
        compiler_params=pltpu.CompilerParams(
            dimension_semantics=("arbitrary",)),
    )(x, codebook, csq)


def _make_sc_gather():
    info = plsc.get_sparse_core_info()
    nc, ns = info.num_cores, info.num_subcores
    nw = nc * ns
    rows_per_w = N // nw
    chunk = 128
    nchunk = rows_per_w // chunk
    mesh = plsc.VectorSubcoreMesh(core_axis_name="c", subcore_axis_name="s")

    @functools.partial(
        pl.kernel, mesh=mesh,
        out_type=jax.ShapeDtypeStruct((N, D), jnp.float32),
        scratch_types=[
            pltpu.VMEM((nchunk, chunk), jnp.int32),
            pltpu.VMEM((chunk, D), jnp.float32),
            pltpu.VMEM((chunk, D), jnp.float32),
            pltpu.SemaphoreType.DMA,
            pltpu.SemaphoreType.DMA,
        ],
    )
    def gather(table_hbm, idx_hbm, out_hbm, idx_v, rows0, rows1, sem0, sem1):
        wid = lax.axis_index("s") * nc + lax.axis_index("c")
        base = wid * rows_per_w
        pltpu.sync_copy(idx_hbm.at[pl.ds(wid * nchunk, nchunk)], idx_v)
        c0 = pltpu.async_copy(table_hbm.at[idx_v.at[0]], rows0, sem0)
        c1 = pltpu.async_copy(table_hbm.at[idx_v.at[1]], rows1, sem1)
        c0.wait()
        pltpu.sync_copy(rows0, out_hbm.at[pl.ds(base, chunk)])
        c1.wait()
        pltpu.sync_copy(rows1, out_hbm.at[pl.ds(base + chunk, chunk)])

    return gather


_sc_gather = None


def _straight_through_body(x_ref, q_ref, o_ref):
    x = x_ref[...]
    o_ref[...] = x + (q_ref[...] - x)


def _straight_through(x, q):
    blk = 1024
    return pl.pallas_call(
        _straight_through_body,
        grid=(N // blk,),
        in_specs=[
            pl.BlockSpec((blk, D), lambda i: (i, 0)),
            pl.BlockSpec((blk, D), lambda i: (i, 0)),
        ],
        out_specs=pl.BlockSpec((blk, D), lambda i: (i, 0)),
        out_shape=jax.ShapeDtypeStruct((N, D), jnp.float32),
    )(x, q)


def kernel(inputs, codebook):
    global _sc_gather
    if _sc_gather is None:
        _sc_gather = _make_sc_gather()
    x = jnp.transpose(inputs, (0, 2, 3, 1)).reshape(N, D)
    csq = jnp.sum(codebook ** 2, axis=1).reshape(1, NK)
    idx3, dmin3 = _dist_argmin(x, codebook, csq)
    idx_flat = idx3.reshape(N)
    q = _sc_gather(codebook, idx_flat.reshape(N // 128, 128))
    qst = _straight_through(x, q)
    e = jnp.sum(dmin3) / float(N * D)
    vq_loss = e + 0.25 * e
    out = jnp.transpose(qst.reshape(8, 32, 32, D), (0, 3, 1, 2))
    return out, vq_loss, idx_flat.reshape(8, 32, 32)


# speedup vs baseline: 1.2583x; 1.0645x over previous
import functools

import jax
import jax.numpy as jnp
from jax import lax
from jax.experimental import pallas as pl
from jax.experimental.pallas import tpu as pltpu
from jax.experimental.pallas import tpu_sc as plsc

NK = 8192
D = 256
N = 8192
TM = 256
GRID = N // TM


def _dist_argmin_body(x_ref, cb_ref, csq_ref, kio_ref, idx_ref, dmin_ref):
    x = x_ref[...]
    xsq = jnp.sum(x * x, axis=1, keepdims=True)
    m2 = lax.dot_general(x + x, cb_ref[...], (((1,), (1,)), ((), ())),
                         preferred_element_type=jnp.float32)
    d = (xsq + csq_ref[...]) - m2
    dmin = jnp.min(d, axis=1, keepdims=True)
    kio = kio_ref[...]
    idxf = jnp.min(jnp.where(d == dmin, kio, float(NK)), axis=1)
    idx_ref[0, 0, :] = idxf.astype(jnp.int32)
    dmin_ref[0, 0, :] = dmin[:, 0]


def _dist_argmin(x, codebook, csq, kio):
    return pl.pallas_call(
        _dist_argmin_body,
        grid=(GRID,),
        in_specs=[
            pl.BlockSpec((TM, D), lambda i: (i, 0)),
            pl.BlockSpec((NK, D), lambda i: (0, 0)),
            pl.BlockSpec((1, NK), lambda i: (0, 0)),
            pl.BlockSpec((1, NK), lambda i: (0, 0)),
        ],
        out_specs=[
            pl.BlockSpec((1, 1, TM), lambda i: (i, 0, 0)),
            pl.BlockSpec((1, 1, TM), lambda i: (i, 0, 0)),
        ],
        out_shape=[
            jax.ShapeDtypeStruct((GRID, 1, TM), jnp.int32),
            jax.ShapeDtypeStruct((GRID, 1, TM), jnp.float32),
        ],
        compiler_params=pltpu.CompilerParams(
            dimension_semantics=("arbitrary",)),
    )(x, codebook, csq, kio)


def _make_sc_gather():
    info = plsc.get_sparse_core_info()
    nc, ns = info.num_cores, info.num_subcores
    nw = nc * ns
    rows_per_w = N // nw
    chunk = 128
    nchunk = rows_per_w // chunk
    mesh = plsc.VectorSubcoreMesh(core_axis_name="c", subcore_axis_name="s")

    @functools.partial(
        pl.kernel, mesh=mesh,
        out_type=jax.ShapeDtypeStruct((N, D), jnp.float32),
        scratch_types=[
            pltpu.VMEM((nchunk, chunk), jnp.int32),
            pltpu.VMEM((chunk, D), jnp.float32),
            pltpu.VMEM((chunk, D), jnp.float32),
            pltpu.SemaphoreType.DMA,
            pltpu.SemaphoreType.DMA,
        ],
    )
    def gather(table_hbm, idx_hbm, out_hbm, idx_v, rows0, rows1, sem0, sem1):
        wid = lax.axis_index("s") * nc + lax.axis_index("c")
        base = wid * rows_per_w
        pltpu.sync_copy(idx_hbm.at[pl.ds(wid * nchunk, nchunk)], idx_v)
        c0 = pltpu.async_copy(table_hbm.at[idx_v.at[0]], rows0, sem0)
        c1 = pltpu.async_copy(table_hbm.at[idx_v.at[1]], rows1, sem1)
        c0.wait()
        pltpu.sync_copy(rows0, out_hbm.at[pl.ds(base, chunk)])
        c1.wait()
        pltpu.sync_copy(rows1, out_hbm.at[pl.ds(base + chunk, chunk)])

    return gather


_sc_gather = None


def kernel(inputs, codebook):
    global _sc_gather
    if _sc_gather is None:
        _sc_gather = _make_sc_gather()
    x = jnp.transpose(inputs, (0, 2, 3, 1)).reshape(N, D)
    csq = jnp.sum(codebook ** 2, axis=1).reshape(1, NK)
    kio = jnp.arange(NK, dtype=jnp.float32).reshape(1, NK)
    idx3, dmin3 = _dist_argmin(x, codebook, csq, kio)
    idx_flat = idx3.reshape(N)
    q = _sc_gather(codebook, idx_flat.reshape(N // 128, 128))
    e = jnp.sum(dmin3) / float(N * D)
    vq_loss = e + 0.25 * e
    out = jnp.transpose(q.reshape(8, 32, 32, D), (0, 3, 1, 2))
    return out, vq_loss, idx_flat.reshape(8, 32, 32)


# speedup vs baseline: 1.5174x; 1.2059x over previous
import functools

import jax
import jax.numpy as jnp
from jax import lax
from jax.experimental import pallas as pl
from jax.experimental.pallas import tpu as pltpu
from jax.experimental.pallas import tpu_sc as plsc

NK = 8192
D = 256
N = 8192
TM = 512
GRID = N // TM
CH = 128


def _dist_argmin_body(x_ref, cb_ref, csq_ref, idx_ref, dmin_ref):
    x = x_ref[...]
    xsq = jnp.sum(x * x, axis=1, keepdims=True)
    m2 = lax.dot_general(x + x, cb_ref[...], (((1,), (1,)), ((), ())),
                         preferred_element_type=jnp.float32)
    best_v = None
    best_c = None
    for c in range(NK // CH):
        dc = (xsq + csq_ref[:, c * CH:(c + 1) * CH]) - m2[:, c * CH:(c + 1) * CH]
        if c == 0:
            best_v = dc
            best_c = jnp.zeros(dc.shape, jnp.float32)
        else:
            upd = dc < best_v
            best_v = jnp.where(upd, dc, best_v)
            best_c = jnp.where(upd, float(c), best_c)
    lane = lax.broadcasted_iota(jnp.int32, (TM, CH), 1).astype(jnp.float32)
    kvec = best_c * float(CH) + lane
    gmin = jnp.min(best_v, axis=1, keepdims=True)
    idxf = jnp.min(jnp.where(best_v == gmin, kvec, float(NK)), axis=1)
    idx_ref[0, 0, :] = idxf.astype(jnp.int32)
    dmin_ref[0, 0, :] = gmin[:, 0]


def _dist_argmin(x, codebook, csq):
    return pl.pallas_call(
        _dist_argmin_body,
        grid=(GRID,),
        in_specs=[
            pl.BlockSpec((TM, D), lambda i: (i, 0)),
            pl.BlockSpec((NK, D), lambda i: (0, 0)),
            pl.BlockSpec((1, NK), lambda i: (0, 0)),
        ],
        out_specs=[
            pl.BlockSpec((1, 1, TM), lambda i: (i, 0, 0)),
            pl.BlockSpec((1, 1, TM), lambda i: (i, 0, 0)),
        ],
        out_shape=[
            jax.ShapeDtypeStruct((GRID, 1, TM), jnp.int32),
            jax.ShapeDtypeStruct((GRID, 1, TM), jnp.float32),
        ],
        compiler_params=pltpu.CompilerParams(
            dimension_semantics=("arbitrary",)),
    )(x, codebook, csq)


def _make_sc_gather():
    info = plsc.get_sparse_core_info()
    nc, ns = info.num_cores, info.num_subcores
    nw = nc * ns
    rows_per_w = N // nw
    chunk = 128
    nchunk = rows_per_w // chunk
    mesh = plsc.VectorSubcoreMesh(core_axis_name="c", subcore_axis_name="s")

    @functools.partial(
        pl.kernel, mesh=mesh,
        out_type=jax.ShapeDtypeStruct((N, D), jnp.float32),
        scratch_types=[
            pltpu.VMEM((nchunk, chunk), jnp.int32),
            pltpu.VMEM((chunk, D), jnp.float32),
            pltpu.VMEM((chunk, D), jnp.float32),
            pltpu.SemaphoreType.DMA,
            pltpu.SemaphoreType.DMA,
        ],
    )
    def gather(table_hbm, idx_hbm, out_hbm, idx_v, rows0, rows1, sem0, sem1):
        wid = lax.axis_index("s") * nc + lax.axis_index("c")
        base = wid * rows_per_w
        pltpu.sync_copy(idx_hbm.at[pl.ds(wid * nchunk, nchunk)], idx_v)
        c0 = pltpu.async_copy(table_hbm.at[idx_v.at[0]], rows0, sem0)
        c1 = pltpu.async_copy(table_hbm.at[idx_v.at[1]], rows1, sem1)
        c0.wait()
        pltpu.sync_copy(rows0, out_hbm.at[pl.ds(base, chunk)])
        c1.wait()
        pltpu.sync_copy(rows1, out_hbm.at[pl.ds(base + chunk, chunk)])

    return gather


_sc_gather = None


def kernel(inputs, codebook):
    global _sc_gather
    if _sc_gather is None:
        _sc_gather = _make_sc_gather()
    x = jnp.transpose(inputs, (0, 2, 3, 1)).reshape(N, D)
    csq = jnp.sum(codebook ** 2, axis=1).reshape(1, NK)
    idx3, dmin3 = _dist_argmin(x, codebook, csq)
    idx_flat = idx3.reshape(N)
    q = _sc_gather(codebook, idx_flat.reshape(N // 128, 128))
    e = jnp.sum(dmin3) / float(N * D)
    vq_loss = e + 0.25 * e
    out = jnp.transpose(q.reshape(8, 32, 32, D), (0, 3, 1, 2))
    return out, vq_loss, idx_flat.reshape(8, 32, 32)


# speedup vs baseline: 1.6762x; 1.1047x over previous
import functools

import jax
import jax.numpy as jnp
from jax import lax
from jax.experimental import pallas as pl
from jax.experimental.pallas import tpu as pltpu
from jax.experimental.pallas import tpu_sc as plsc

NK = 8192
D = 256
N = 8192
TM = 4096
GRID = N // TM
CH = 128


def _dist_argmin_body(x_ref, cb_ref, csq_ref, idx_ref, dmin_ref):
    x = x_ref[...]
    xsq = jnp.sum(x * x, axis=1, keepdims=True)
    m2 = lax.dot_general(x + x, cb_ref[...], (((1,), (1,)), ((), ())),
                         preferred_element_type=jnp.float32)
    best_v = None
    best_c = None
    for c in range(NK // CH):
        dc = (xsq + csq_ref[:, c * CH:(c + 1) * CH]) - m2[:, c * CH:(c + 1) * CH]
        if c == 0:
            best_v = dc
            best_c = jnp.zeros(dc.shape, jnp.float32)
        else:
            upd = dc < best_v
            best_v = jnp.minimum(dc, best_v)
            best_c = jnp.where(upd, float(c), best_c)
    lane = lax.broadcasted_iota(jnp.int32, (TM, CH), 1).astype(jnp.float32)
    kvec = best_c * float(CH) + lane
    gmin = jnp.min(best_v, axis=1, keepdims=True)
    idxf = jnp.min(jnp.where(best_v == gmin, kvec, float(NK)), axis=1)
    idx_ref[0, 0, :] = idxf.astype(jnp.int32)
    dmin_ref[0, 0, :] = gmin[:, 0]


def _dist_argmin(x, codebook, csq):
    return pl.pallas_call(
        _dist_argmin_body,
        grid=(GRID,),
        in_specs=[
            pl.BlockSpec((TM, D), lambda i: (i, 0)),
            pl.BlockSpec((NK, D), lambda i: (0, 0)),
            pl.BlockSpec((1, NK), lambda i: (0, 0)),
        ],
        out_specs=[
            pl.BlockSpec((1, 1, TM), lambda i: (i, 0, 0)),
            pl.BlockSpec((1, 1, TM), lambda i: (i, 0, 0)),
        ],
        out_shape=[
            jax.ShapeDtypeStruct((GRID, 1, TM), jnp.int32),
            jax.ShapeDtypeStruct((GRID, 1, TM), jnp.float32),
        ],
        compiler_params=pltpu.CompilerParams(
            dimension_semantics=("arbitrary",)),
    )(x, codebook, csq)


def _make_sc_gather():
    info = plsc.get_sparse_core_info()
    nc, ns = info.num_cores, info.num_subcores
    nw = nc * ns
    rows_per_w = N // nw
    chunk = 128
    nchunk = rows_per_w // chunk
    mesh = plsc.VectorSubcoreMesh(core_axis_name="c", subcore_axis_name="s")

    @functools.partial(
        pl.kernel, mesh=mesh,
        out_type=jax.ShapeDtypeStruct((N, D), jnp.float32),
        scratch_types=[
            pltpu.VMEM((nchunk, chunk), jnp.int32),
            pltpu.VMEM((chunk, D), jnp.float32),
            pltpu.VMEM((chunk, D), jnp.float32),
            pltpu.SemaphoreType.DMA,
            pltpu.SemaphoreType.DMA,
        ],
    )
    def gather(table_hbm, idx_hbm, out_hbm, idx_v, rows0, rows1, sem0, sem1):
        wid = lax.axis_index("s") * nc + lax.axis_index("c")
        base = wid * rows_per_w
        pltpu.sync_copy(idx_hbm.at[pl.ds(wid * nchunk, nchunk)], idx_v)
        c0 = pltpu.async_copy(table_hbm.at[idx_v.at[0]], rows0, sem0)
        c1 = pltpu.async_copy(table_hbm.at[idx_v.at[1]], rows1, sem1)
        c0.wait()
        pltpu.sync_copy(rows0, out_hbm.at[pl.ds(base, chunk)])
        c1.wait()
        pltpu.sync_copy(rows1, out_hbm.at[pl.ds(base + chunk, chunk)])

    return gather


_sc_gather = None


def kernel(inputs, codebook):
    global _sc_gather
    if _sc_gather is None:
        _sc_gather = _make_sc_gather()
    x = jnp.transpose(inputs, (0, 2, 3, 1)).reshape(N, D)
    csq = jnp.sum(codebook ** 2, axis=1).reshape(1, NK)
    idx3, dmin3 = _dist_argmin(x, codebook, csq)
    idx_flat = idx3.reshape(N)
    q = _sc_gather(codebook, idx_flat.reshape(N // 128, 128))
    e = jnp.sum(dmin3) / float(N * D)
    vq_loss = e + 0.25 * e
    out = jnp.transpose(q.reshape(8, 32, 32, D), (0, 3, 1, 2))
    return out, vq_loss, idx_flat.reshape(8, 32, 32)
